# baseline (speedup 1.0000x reference)
import functools

import numpy as np
import jax
import jax.numpy as jnp
from jax.experimental import pallas as pl
from jax.experimental.pallas import tpu as pltpu

F32 = jnp.float32
BF16 = jnp.bfloat16
_HI = jax.lax.Precision.HIGHEST

_B = 32
_T = 64
_TQ = 16
_NB = 512
_CD = 256


def _sep_perms():
    pos0, rot0, vel0, foot0, nch = 4, 67, 193, 259, 263
    lower = np.array([0, 1, 2, 4, 5, 7, 8, 10, 11])
    lower_ex = lower[1:] - 1
    upper = np.array([3, 6, 9, 12, 13, 14, 15, 16, 17, 18, 19, 20, 21])
    upper_ex = upper - 1
    up = np.concatenate([
        (pos0 + upper_ex[:, None] * 3 + np.arange(3)).ravel(),
        (rot0 + upper_ex[:, None] * 6 + np.arange(6)).ravel(),
        (vel0 + upper[:, None] * 3 + np.arange(3)).ravel(),
    ])
    low = np.concatenate([
        np.arange(4),
        (pos0 + lower_ex[:, None] * 3 + np.arange(3)).ravel(),
        (rot0 + lower_ex[:, None] * 6 + np.arange(6)).ravel(),
        (vel0 + lower[:, None] * 3 + np.arange(3)).ravel(),
        np.arange(foot0, nch),
    ])
    return up, low


_PERM_UP, _PERM_LOW = _sep_perms()


def _taps(w):
    return jnp.transpose(w, (2, 1, 0)).astype(BF16)


def _bias(b):
    return b.reshape(1, -1)


def _dotb(x, w):
    return jax.lax.dot(x, w, preferred_element_type=F32)


def _conv(x, wt, b, dil, padn):
    bq, tq, c = x.shape
    k = wt.shape[0]
    co = wt.shape[2]
    xp = jnp.pad(x, ((0, 0), (padn, padn), (0, 0))) if padn else x
    xb = xp.astype(BF16)
    acc = None
    for i in range(k):
        s = jax.lax.slice(xb, (0, i * dil, 0), (bq, i * dil + tq, c))
        y = _dotb(s.reshape(bq * tq, c), wt[i])
        acc = y if acc is None else acc + y
    return (acc + b).reshape(bq, tq, co)


def _down(x, wt, b):
    bq, tq, c = x.shape
    to = tq // 2
    xb = jnp.pad(x, ((0, 0), (1, 1), (0, 0))).astype(BF16)
    r = xb.reshape(bq, to + 1, 2, c)
    parts = (r[:, 0:to, 0, :], r[:, 0:to, 1, :],
             r[:, 1:, 0, :], r[:, 1:, 1, :])
    acc = None
    for i in range(4):
        y = _dotb(parts[i].reshape(bq * to, c), wt[i])
        acc = y if acc is None else acc + y
    return (acc + b).reshape(bq, to, wt.shape[2])


def _enc_fwd(x, ws):
    it = iter(ws)
    h = _conv(x, next(it), next(it), 1, 1)
    h = jnp.maximum(h, 0.0)
    for _ in range(2):
        h = _down(h, next(it), next(it))
        for j in range(3):
            w1, b1, w2, b2 = next(it), next(it), next(it), next(it)
            g = jnp.maximum(h, 0.0)
            g = _conv(g, w1, b1, 3 ** j, 3 ** j)
            g = jnp.maximum(g, 0.0)
            g = _conv(g, w2, b2, 1, 0)
            h = h + g
    h = _conv(h, next(it), next(it), 1, 1)
    return h


def _quant(xf, cb, cbt):
    r, n = xf.shape[0], cb.shape[0]
    x2 = jnp.sum(xf * xf, axis=1, keepdims=True)
    c2 = jnp.sum(cb * cb, axis=1)
    xy = _dotb(xf.astype(BF16), cbt)
    dist = (x2 - 2.0 * xy) + c2[None, :]
    m = jnp.min(dist, axis=1, keepdims=True)
    iota = jax.lax.broadcasted_iota(jnp.int32, (r, n), 1)
    idx = jnp.min(jnp.where(dist == m, iota, n), axis=1)
    oh = (iota == idx[:, None]).astype(F32)
    xd = jax.lax.dot(oh, cb, precision=_HI, preferred_element_type=F32)
    counts = jnp.sum(oh, axis=0)
    prob = counts / r
    ppl = jnp.exp(-jnp.sum(prob * jnp.log(prob + 1e-7)))
    commit = jnp.mean((xf - xd) ** 2)
    return xd, commit, ppl


def _enc_body(nw, *refs):
    x = refs[0][...]
    ws = [rr[...] for rr in refs[1:1 + nw]]
    cb = refs[1 + nw][...]
    cbt = refs[2 + nw][...]
    xd_ref, com_ref, ppl_ref = refs[3 + nw:6 + nw]
    h = _enc_fwd(x, ws)
    xf = h.reshape(_B * _TQ, _CD)
    xd, commit, ppl = _quant(xf, cb, cbt)
    xd_ref[...] = xd
    com_ref[...] = commit.reshape(1, 1)
    ppl_ref[...] = ppl.reshape(1, 1)


def _dec_fwd(x, ws):
    it = iter(ws)
    h = _conv(x, next(it), next(it), 1, 1)
    h = jnp.maximum(h, 0.0)
    for _ in range(2):
        for j in range(3):
            w1, b1, w2, b2 = next(it), next(it), next(it), next(it)
            d = 3 ** (2 - j)
            g = jnp.maximum(h, 0.0)
            g = _conv(g, w1, b1, d, d)
            g = jnp.maximum(g, 0.0)
            g = _conv(g, w2, b2, 1, 0)
            h = h + g
        wu, bu = next(it), next(it)
        bq, tq, c = h.shape
        hr = jnp.stack([h, h], axis=2).reshape(bq, 2 * tq, c)
        h = _conv(hr, wu, bu, 1, 1)
    h = _conv(h, next(it), next(it), 1, 1)
    h = jnp.maximum(h, 0.0)
    h = _conv(h, next(it), next(it), 1, 1)
    return h


def _dec_body(nw, *refs):
    x = refs[0][...]
    ws = [rr[...] for rr in refs[1:1 + nw]]
    out_ref = refs[1 + nw]
    out_ref[...] = _dec_fwd(x, ws)


def _enc_wlist(p, perm):
    w_in = p['conv_in']['w']
    w_in = jnp.zeros((w_in.shape[0], 263, 3), F32).at[:, perm, :].set(w_in)
    out = [_taps(w_in), _bias(p['conv_in']['b'])]
    for blk in p['downs']:
        out += [_taps(blk['down']['w']), _bias(blk['down']['b'])]
        for rb in blk['res']:
            out += [_taps(rb['c1']['w']), _bias(rb['c1']['b']),
                    _taps(rb['c2']['w']), _bias(rb['c2']['b'])]
    out += [_taps(p['conv_out']['w']), _bias(p['conv_out']['b'])]
    return out


def _dec_wlist(p):
    out = [_taps(p['conv_in']['w']), _bias(p['conv_in']['b'])]
    for blk in p['ups']:
        for rb in blk['res']:
            out += [_taps(rb['c1']['w']), _bias(rb['c1']['b']),
                    _taps(rb['c2']['w']), _bias(rb['c2']['b'])]
        out += [_taps(blk['up']['w']), _bias(blk['up']['b'])]
    out += [_taps(p['conv_mid']['w']), _bias(p['conv_mid']['b']),
            _taps(p['conv_out']['w']), _bias(p['conv_out']['b'])]
    return out


def _enc_call(x, wlist, cb):
    cbt = jnp.transpose(cb).astype(BF16)
    nw = len(wlist)
    out_shape = (jax.ShapeDtypeStruct((_B * _TQ, _CD), F32),
                 jax.ShapeDtypeStruct((1, 1), F32),
                 jax.ShapeDtypeStruct((1, 1), F32))
    return pl.pallas_call(
        functools.partial(_enc_body, nw), out_shape=out_shape,
    )(x, *wlist, cb, cbt)


def _dec_call(xq, wlist):
    nw = len(wlist)
    out_shape = jax.ShapeDtypeStruct((_B, _T, 263), F32)
    return pl.pallas_call(
        functools.partial(_dec_body, nw), out_shape=out_shape,
    )(xq, *wlist)


def kernel(x, params):
    wu = _enc_wlist(params['enc_up'], _PERM_UP)
    wl = _enc_wlist(params['enc_low'], _PERM_LOW)
    xdu, cu, _ = _enc_call(x, wu, params['cb_up'])
    xdl, cl, ppl = _enc_call(x, wl, params['cb_low'])
    xq = jnp.concatenate([xdu, xdl], axis=-1).reshape(_B, _TQ, 2 * _CD)
    xout = _dec_call(xq, _dec_wlist(params['dec']))
    loss = (cu + cl)[0, 0]
    return xout, loss, ppl[0, 0]


# speedup vs baseline: 1.0427x; 1.0427x over previous
import functools

import numpy as np
import jax
import jax.numpy as jnp
from jax.experimental import pallas as pl
from jax.experimental.pallas import tpu as pltpu

F32 = jnp.float32
BF16 = jnp.bfloat16
_HI = jax.lax.Precision.HIGHEST

_B = 32
_T = 64
_TQ = 16
_NB = 512
_CD = 256


def _sep_perms():
    pos0, rot0, vel0, foot0, nch = 4, 67, 193, 259, 263
    lower = np.array([0, 1, 2, 4, 5, 7, 8, 10, 11])
    lower_ex = lower[1:] - 1
    upper = np.array([3, 6, 9, 12, 13, 14, 15, 16, 17, 18, 19, 20, 21])
    upper_ex = upper - 1
    up = np.concatenate([
        (pos0 + upper_ex[:, None] * 3 + np.arange(3)).ravel(),
        (rot0 + upper_ex[:, None] * 6 + np.arange(6)).ravel(),
        (vel0 + upper[:, None] * 3 + np.arange(3)).ravel(),
    ])
    low = np.concatenate([
        np.arange(4),
        (pos0 + lower_ex[:, None] * 3 + np.arange(3)).ravel(),
        (rot0 + lower_ex[:, None] * 6 + np.arange(6)).ravel(),
        (vel0 + lower[:, None] * 3 + np.arange(3)).ravel(),
        np.arange(foot0, nch),
    ])
    return up, low


_PERM_UP, _PERM_LOW = _sep_perms()


def _taps(w):
    return jnp.transpose(w, (2, 1, 0)).astype(BF16)


def _bias(b):
    return b.reshape(1, -1)


def _dotb(x, w):
    return jax.lax.dot(x, w, preferred_element_type=F32)


def _conv(x, wt, b, dil, padn):
    tq, bq, c = x.shape
    k = wt.shape[0]
    co = wt.shape[2]
    xb = x.astype(BF16)
    xp = jnp.pad(xb, ((padn, padn), (0, 0), (0, 0))) if padn else xb
    acc = None
    for i in range(k):
        s = jax.lax.slice(xp, (i * dil, 0, 0), (i * dil + tq, bq, c))
        y = _dotb(s.reshape(tq * bq, c), wt[i])
        acc = y if acc is None else acc + y
    return (acc + b).reshape(tq, bq, co)


def _down(x, wt, b):
    tq, bq, c = x.shape
    to = tq // 2
    xb = jnp.pad(x.astype(BF16), ((1, 1), (0, 0), (0, 0)))
    r = xb.reshape(to + 1, 2, bq, c)
    parts = (r[0:to, 0], r[0:to, 1], r[1:, 0], r[1:, 1])
    acc = None
    for i in range(4):
        y = _dotb(parts[i].reshape(to * bq, c), wt[i])
        acc = y if acc is None else acc + y
    return (acc + b).reshape(to, bq, wt.shape[2])


def _enc_fwd(x, ws):
    it = iter(ws)
    h = _conv(x, next(it), next(it), 1, 1)
    h = jnp.maximum(h, 0.0)
    for _ in range(2):
        h = _down(h, next(it), next(it))
        for j in range(3):
            w1, b1, w2, b2 = next(it), next(it), next(it), next(it)
            g = jnp.maximum(h, 0.0)
            g = _conv(g, w1, b1, 3 ** j, 3 ** j)
            g = jnp.maximum(g, 0.0)
            g = _conv(g, w2, b2, 1, 0)
            h = h + g
    h = _conv(h, next(it), next(it), 1, 1)
    return h


def _quant(xf, cb, cbt):
    r, n = xf.shape[0], cb.shape[0]
    x2 = jnp.sum(xf * xf, axis=1, keepdims=True)
    c2 = jnp.sum(cb * cb, axis=1)
    xy = _dotb(xf.astype(BF16), cbt)
    dist = (x2 - 2.0 * xy) + c2[None, :]
    m = jnp.min(dist, axis=1, keepdims=True)
    iota = jax.lax.broadcasted_iota(jnp.int32, (r, n), 1)
    idx = jnp.min(jnp.where(dist == m, iota, n), axis=1)
    oh = (iota == idx[:, None]).astype(F32)
    xd = jax.lax.dot(oh, cb, precision=_HI, preferred_element_type=F32)
    counts = jnp.sum(oh, axis=0)
    prob = counts / r
    ppl = jnp.exp(-jnp.sum(prob * jnp.log(prob + 1e-7)))
    commit = jnp.mean((xf - xd) ** 2)
    return xd, commit, ppl


def _enc_body(nw, *refs):
    x = refs[0][...]
    ws = [rr[...] for rr in refs[1:1 + nw]]
    cb = refs[1 + nw][...]
    cbt = refs[2 + nw][...]
    xd_ref, com_ref, ppl_ref = refs[3 + nw:6 + nw]
    h = _enc_fwd(x, ws)
    xf = h.reshape(_TQ * _B, _CD)
    xd, commit, ppl = _quant(xf, cb, cbt)
    xd_ref[...] = xd
    com_ref[...] = commit.reshape(1, 1)
    ppl_ref[...] = ppl.reshape(1, 1)


def _dec_fwd(x, ws):
    it = iter(ws)
    h = _conv(x, next(it), next(it), 1, 1)
    h = jnp.maximum(h, 0.0)
    for _ in range(2):
        for j in range(3):
            w1, b1, w2, b2 = next(it), next(it), next(it), next(it)
            d = 3 ** (2 - j)
            g = jnp.maximum(h, 0.0)
            g = _conv(g, w1, b1, d, d)
            g = jnp.maximum(g, 0.0)
            g = _conv(g, w2, b2, 1, 0)
            h = h + g
        wu, bu = next(it), next(it)
        tq, bq, c = h.shape
        hr = jnp.stack([h, h], axis=1).reshape(2 * tq, bq, c)
        h = _conv(hr, wu, bu, 1, 1)
    h = _conv(h, next(it), next(it), 1, 1)
    h = jnp.maximum(h, 0.0)
    h = _conv(h, next(it), next(it), 1, 1)
    return h


def _dec_body(nw, *refs):
    x = refs[0][...]
    ws = [rr[...] for rr in refs[1:1 + nw]]
    out_ref = refs[1 + nw]
    out_ref[...] = _dec_fwd(x, ws)


def _enc_wlist(p, perm):
    w_in = p['conv_in']['w']
    w_in = jnp.zeros((w_in.shape[0], 263, 3), F32).at[:, perm, :].set(w_in)
    out = [_taps(w_in), _bias(p['conv_in']['b'])]
    for blk in p['downs']:
        out += [_taps(blk['down']['w']), _bias(blk['down']['b'])]
        for rb in blk['res']:
            out += [_taps(rb['c1']['w']), _bias(rb['c1']['b']),
                    _taps(rb['c2']['w']), _bias(rb['c2']['b'])]
    out += [_taps(p['conv_out']['w']), _bias(p['conv_out']['b'])]
    return out


def _dec_wlist(p):
    out = [_taps(p['conv_in']['w']), _bias(p['conv_in']['b'])]
    for blk in p['ups']:
        for rb in blk['res']:
            out += [_taps(rb['c1']['w']), _bias(rb['c1']['b']),
                    _taps(rb['c2']['w']), _bias(rb['c2']['b'])]
        out += [_taps(blk['up']['w']), _bias(blk['up']['b'])]
    out += [_taps(p['conv_mid']['w']), _bias(p['conv_mid']['b']),
            _taps(p['conv_out']['w']), _bias(p['conv_out']['b'])]
    return out


def _enc_call(x, wlist, cb):
    cbt = jnp.transpose(cb).astype(BF16)
    nw = len(wlist)
    out_shape = (jax.ShapeDtypeStruct((_TQ * _B, _CD), F32),
                 jax.ShapeDtypeStruct((1, 1), F32),
                 jax.ShapeDtypeStruct((1, 1), F32))
    return pl.pallas_call(
        functools.partial(_enc_body, nw), out_shape=out_shape,
    )(x, *wlist, cb, cbt)


def _dec_call(xq, wlist):
    nw = len(wlist)
    out_shape = jax.ShapeDtypeStruct((_T, _B, 263), F32)
    return pl.pallas_call(
        functools.partial(_dec_body, nw), out_shape=out_shape,
    )(xq, *wlist)


def kernel(x, params):
    xt = jnp.transpose(x, (1, 0, 2))
    wu = _enc_wlist(params['enc_up'], _PERM_UP)
    wl = _enc_wlist(params['enc_low'], _PERM_LOW)
    xdu, cu, _ = _enc_call(xt, wu, params['cb_up'])
    xdl, cl, ppl = _enc_call(xt, wl, params['cb_low'])
    xq = jnp.concatenate([xdu, xdl], axis=-1).reshape(_TQ, _B, 2 * _CD)
    xout = _dec_call(xq, _dec_wlist(params['dec']))
    loss = (cu + cl)[0, 0]
    return jnp.transpose(xout, (1, 0, 2)), loss, ppl[0, 0]
